# Initial kernel scaffold; baseline (speedup 1.0000x reference)
import functools

import jax
import jax.numpy as jnp
from jax import lax
from jax.experimental import pallas as pl
from jax.experimental.pallas import tpu as pltpu
from jax.experimental.pallas import tpu_sc as plsc

N = 50000
E = 800000
D = 64
H = 128
W = 32

NC = 2
NS = 16
C = 80
EPT = E // NS
NITER = EPT // C
CNT_SPLIT = NITER // 2
RPT = N // NS
DR = 625
NDR = RPT // DR

BN = 1000
GRID = N // BN


def _agg_layer1(t0, t1, src, dst, z32, z8, o8):
    mesh = plsc.VectorSubcoreMesh(core_axis_name="c", subcore_axis_name="s")

    @functools.partial(
        pl.kernel,
        mesh=mesh,
        out_type=[
            jax.ShapeDtypeStruct((N, W), jnp.float32),
            jax.ShapeDtypeStruct((N, W), jnp.float32),
            jax.ShapeDtypeStruct((NC, N, 8), jnp.float32),
        ],
        scratch_types=[
            pltpu.VMEM((C,), jnp.int32),
            pltpu.VMEM((C,), jnp.int32),
            pltpu.VMEM((C, W), jnp.float32),
            pltpu.VMEM((C, 8), jnp.float32),
            pltpu.VMEM((DR, W), jnp.float32),
            pltpu.VMEM((DR, W), jnp.float32),
            pltpu.VMEM((DR, 8), jnp.float32),
            pltpu.VMEM_SHARED((N, W), jnp.float32),
            pltpu.VMEM_SHARED((N, 8), jnp.float32),
            pltpu.SemaphoreType.DMA,
        ],
    )
    def k(t0_h, t1_h, src_h, dst_h, z32_h, z8_h, o8_h,
          out0_h, out1_h, cnt_h,
          sidx, didx, rows, ones, zbuf, dbuf, cbuf, acc, cacc, gsem):
        c = lax.axis_index("c")
        s = lax.axis_index("s")
        pltpu.sync_copy(o8_h, ones)
        pltpu.sync_copy(z32_h, zbuf)
        pltpu.sync_copy(z8_h, cbuf)
        for j in range(NDR):
            r0 = s * RPT + j * DR
            pltpu.sync_copy(zbuf, acc.at[pl.ds(r0, DR)])
            pltpu.sync_copy(cbuf, cacc.at[pl.ds(r0, DR)])
        plsc.subcore_barrier()

        tables = (t0_h, t1_h)
        outs = (out0_h, out1_h)
        for cc in range(NC):
            @pl.when(c == cc)
            def _():
                tbl_h = tables[cc]
                out_h = outs[cc]
                ebase = s * EPT

                def body(i, carry):
                    off = ebase + i * C
                    pltpu.sync_copy(src_h.at[pl.ds(off, C)], sidx)
                    pltpu.sync_copy(dst_h.at[pl.ds(off, C)], didx)
                    pltpu.async_copy(tbl_h.at[sidx], rows, gsem).wait()
                    pltpu.sync_copy(rows, acc.at[didx], add=True)
                    counts_here = (i < CNT_SPLIT) if cc == 0 else (i >= CNT_SPLIT)

                    @pl.when(counts_here)
                    def _():
                        pltpu.sync_copy(ones, cacc.at[didx], add=True)
                    return carry

                lax.fori_loop(0, NITER, body, 0)
                plsc.subcore_barrier()
                for j in range(NDR):
                    r0 = s * RPT + j * DR
                    pltpu.sync_copy(acc.at[pl.ds(r0, DR)], dbuf)
                    pltpu.sync_copy(dbuf, out_h.at[pl.ds(r0, DR)])
                    pltpu.sync_copy(cacc.at[pl.ds(r0, DR)], cbuf)
                    pltpu.sync_copy(cbuf, cnt_h.at[cc, pl.ds(r0, DR)])

    return k(t0, t1, src, dst, z32, z8, o8)


def _agg_layer2(h0, h1, h2, h3, src, dst, z32):
    mesh = plsc.VectorSubcoreMesh(core_axis_name="c", subcore_axis_name="s")

    @functools.partial(
        pl.kernel,
        mesh=mesh,
        out_type=[jax.ShapeDtypeStruct((N, W), jnp.float32) for _ in range(4)],
        scratch_types=[
            pltpu.VMEM((C,), jnp.int32),
            pltpu.VMEM((C,), jnp.int32),
            pltpu.VMEM((C, W), jnp.float32),
            pltpu.VMEM((DR, W), jnp.float32),
            pltpu.VMEM((DR, W), jnp.float32),
            pltpu.VMEM_SHARED((N, W), jnp.float32),
            pltpu.SemaphoreType.DMA,
        ],
    )
    def k(h0_h, h1_h, h2_h, h3_h, src_h, dst_h, z32_h,
          g0_h, g1_h, g2_h, g3_h,
          sidx, didx, rows, zbuf, dbuf, acc, gsem):
        c = lax.axis_index("c")
        s = lax.axis_index("s")
        pltpu.sync_copy(z32_h, zbuf)
        tables = (h0_h, h1_h, h2_h, h3_h)
        outs = (g0_h, g1_h, g2_h, g3_h)
        for cc in range(NC):
            @pl.when(c == cc)
            def _():
                for rnd in range(2):
                    ci = cc + 2 * rnd
                    tbl_h = tables[ci]
                    out_h = outs[ci]
                    for j in range(NDR):
                        pltpu.sync_copy(zbuf, acc.at[pl.ds(s * RPT + j * DR, DR)])
                    plsc.subcore_barrier()
                    ebase = s * EPT

                    def body(i, carry):
                        off = ebase + i * C
                        pltpu.sync_copy(src_h.at[pl.ds(off, C)], sidx)
                        pltpu.sync_copy(dst_h.at[pl.ds(off, C)], didx)
                        pltpu.async_copy(tbl_h.at[sidx], rows, gsem).wait()
                        pltpu.sync_copy(rows, acc.at[didx], add=True)
                        return carry

                    lax.fori_loop(0, NITER, body, 0)
                    plsc.subcore_barrier()
                    for j in range(NDR):
                        r0 = s * RPT + j * DR
                        pltpu.sync_copy(acc.at[pl.ds(r0, DR)], dbuf)
                        pltpu.sync_copy(dbuf, out_h.at[pl.ds(r0, DR)])

    return k(h0, h1, h2, h3, src, dst, z32)


def _dense1(a0, a1, c0, c1, x, W1l, b1l, W1r):
    def body(a0_r, a1_r, c0_r, c1_r, x_r, wl_r, bl_r, wr_r,
             o0, o1, o2, o3, rc):
        cnt = c0_r[:, 0:1] + c1_r[:, 0:1]
        recip = 1.0 / jnp.maximum(cnt, 1.0)
        m = lax.dot_general(a0_r[...], wl_r[:, :W],
                            (((1,), (1,)), ((), ())),
                            preferred_element_type=jnp.float32)
        m += lax.dot_general(a1_r[...], wl_r[:, W:],
                             (((1,), (1,)), ((), ())),
                             preferred_element_type=jnp.float32)
        sf = lax.dot_general(x_r[...], wr_r[...],
                             (((1,), (1,)), ((), ())),
                             preferred_element_type=jnp.float32)
        h = jnp.maximum(m * recip + bl_r[...] + sf, 0.0)
        o0[...] = h[:, 0:W]
        o1[...] = h[:, W:2 * W]
        o2[...] = h[:, 2 * W:3 * W]
        o3[...] = h[:, 3 * W:4 * W]
        rc[...] = jnp.broadcast_to(recip, (BN, 8))

    node = lambda w: pl.BlockSpec((BN, w), lambda i: (i, 0))
    full = lambda a, b: pl.BlockSpec((a, b), lambda i: (0, 0))
    return pl.pallas_call(
        body,
        grid=(GRID,),
        in_specs=[node(W), node(W), node(8), node(8), node(D),
                  full(H, D), full(1, H), full(H, D)],
        out_specs=[node(W), node(W), node(W), node(W), node(8)],
        out_shape=[jax.ShapeDtypeStruct((N, W), jnp.float32) for _ in range(4)]
        + [jax.ShapeDtypeStruct((N, 8), jnp.float32)],
    )(a0, a1, c0, c1, x, W1l, b1l, W1r)


def _dense2(g0, g1, g2, g3, h0, h1, h2, h3, rc, W2l, b2l, W2r, Wc, bc):
    def body(g0_r, g1_r, g2_r, g3_r, h0_r, h1_r, h2_r, h3_r, rc_r,
             wl_r, bl_r, wr_r, wc_r, bc_r, o):
        gs = (g0_r, g1_r, g2_r, g3_r)
        hs = (h0_r, h1_r, h2_r, h3_r)
        m = jnp.zeros((BN, H), jnp.float32)
        sf = jnp.zeros((BN, H), jnp.float32)
        for i in range(4):
            m += lax.dot_general(gs[i][...], wl_r[:, i * W:(i + 1) * W],
                                 (((1,), (1,)), ((), ())),
                                 preferred_element_type=jnp.float32)
            sf += lax.dot_general(hs[i][...], wr_r[:, i * W:(i + 1) * W],
                                  (((1,), (1,)), ((), ())),
                                  preferred_element_type=jnp.float32)
        h = jnp.maximum(m * rc_r[:, 0:1] + bl_r[...] + sf, 0.0)
        logit = lax.dot_general(h, wc_r[...], (((1,), (1,)), ((), ())),
                                preferred_element_type=jnp.float32) + bc_r[...]
        o[...] = jax.nn.sigmoid(logit)

    node = lambda w: pl.BlockSpec((BN, w), lambda i: (i, 0))
    full = lambda a, b: pl.BlockSpec((a, b), lambda i: (0, 0))
    return pl.pallas_call(
        body,
        grid=(GRID,),
        in_specs=[node(W)] * 8 + [node(8),
                  full(H, H), full(1, H), full(H, H), full(1, H), full(1, 1)],
        out_specs=node(1),
        out_shape=jax.ShapeDtypeStruct((N, 1), jnp.float32),
    )(g0, g1, g2, g3, h0, h1, h2, h3, rc, W2l, b2l, W2r, Wc, bc)


def kernel(x, edge_index, W1l, b1l, W1r, W2l, b2l, W2r, Wc, bc):
    src = edge_index[0]
    dst = edge_index[1]
    t0 = x[:, :W]
    t1 = x[:, W:]
    z32 = jnp.zeros((DR, W), jnp.float32)
    z8 = jnp.zeros((DR, 8), jnp.float32)
    o8 = jnp.ones((C, 8), jnp.float32)

    a0, a1, cnt = _agg_layer1(t0, t1, src, dst, z32, z8, o8)
    h0, h1, h2, h3, rc = _dense1(a0, a1, cnt[0], cnt[1], x,
                                 W1l, b1l.reshape(1, H), W1r)
    g0, g1, g2, g3 = _agg_layer2(h0, h1, h2, h3, src, dst, z32)
    return _dense2(g0, g1, g2, g3, h0, h1, h2, h3, rc,
                   W2l, b2l.reshape(1, H), W2r, Wc, bc.reshape(1, 1))


# speedup vs baseline: 2.2385x; 2.2385x over previous
import functools

import jax
import jax.numpy as jnp
from jax import lax
from jax.experimental import pallas as pl
from jax.experimental.pallas import tpu as pltpu
from jax.experimental.pallas import tpu_sc as plsc

N = 50000
NP = 50176
E = 800000
D = 64
H = 128
W = 32

NC = 2
NS = 16
C = 80
EPT = E // NS
NITER = EPT // C
RPT = NP // NS
DR = 392
NDR = RPT // DR

CW = 8
CC_ = 40
CEPW = E // (NC * NS)
CNITER = CEPW // CC_

BN = 784
GRID = NP // BN

_mesh = lambda: plsc.VectorSubcoreMesh(core_axis_name="c", subcore_axis_name="s")


def _counts(dst, z8, o8):
    @functools.partial(
        pl.kernel,
        mesh=_mesh(),
        compiler_params=pltpu.CompilerParams(use_tc_tiling_on_sc=False),
        out_type=jax.ShapeDtypeStruct((NC, NP, CW), jnp.float32),
        scratch_types=[
            pltpu.VMEM((CC_,), jnp.int32),
            pltpu.VMEM((CC_, CW), jnp.float32),
            pltpu.VMEM((DR, CW), jnp.float32),
            pltpu.VMEM_SHARED((NP, CW), jnp.float32),
        ],
    )
    def k(dst_h, z8_h, o8_h, cnt_h, didx, ones, cbuf, cacc):
        c = lax.axis_index("c")
        s = lax.axis_index("s")
        pltpu.sync_copy(o8_h, ones)
        pltpu.sync_copy(z8_h, cbuf)
        for j in range(NDR):
            pltpu.sync_copy(cbuf, cacc.at[pl.ds(s * RPT + j * DR, DR)])
        plsc.subcore_barrier()
        for cc in range(NC):
            @pl.when(c == cc)
            def _():
                ebase = s * EPT + cc * CEPW

                def body(i, carry):
                    pltpu.sync_copy(dst_h.at[pl.ds(ebase + i * CC_, CC_)], didx)
                    pltpu.sync_copy(ones, cacc.at[didx], add=True)
                    return carry

                lax.fori_loop(0, CNITER, body, 0)
                plsc.subcore_barrier()
                for j in range(NDR):
                    r0 = s * RPT + j * DR
                    pltpu.sync_copy(cacc.at[pl.ds(r0, DR)], cbuf)
                    pltpu.sync_copy(cbuf, cnt_h.at[cc, pl.ds(r0, DR)])

    return k(dst, z8, o8)


def _agg(tables, src, dst, z32):
    ntab = len(tables)
    rounds = ntab // NC

    @functools.partial(
        pl.kernel,
        mesh=_mesh(),
        compiler_params=pltpu.CompilerParams(use_tc_tiling_on_sc=False),
        out_type=[jax.ShapeDtypeStruct((NP, W), jnp.float32)
                  for _ in range(ntab)],
        scratch_types=[
            pltpu.VMEM((C,), jnp.int32),
            pltpu.VMEM((C,), jnp.int32),
            pltpu.VMEM((C, W), jnp.float32),
            pltpu.VMEM((DR, W), jnp.float32),
            pltpu.VMEM((DR, W), jnp.float32),
            pltpu.VMEM_SHARED((NP, W), jnp.float32),
            pltpu.SemaphoreType.DMA,
        ],
    )
    def k(*refs):
        tbls = refs[:ntab]
        src_h, dst_h, z32_h = refs[ntab:ntab + 3]
        outs = refs[ntab + 3:2 * ntab + 3]
        sidx, didx, rows, zbuf, dbuf, acc, gsem = refs[2 * ntab + 3:]
        c = lax.axis_index("c")
        s = lax.axis_index("s")
        pltpu.sync_copy(z32_h, zbuf)
        for cc in range(NC):
            @pl.when(c == cc)
            def _():
                for rnd in range(rounds):
                    ci = cc + NC * rnd
                    tbl_h = tbls[ci]
                    out_h = outs[ci]
                    for j in range(NDR):
                        pltpu.sync_copy(zbuf, acc.at[pl.ds(s * RPT + j * DR, DR)])
                    plsc.subcore_barrier()
                    ebase = s * EPT

                    def body(i, carry):
                        off = ebase + i * C
                        pltpu.sync_copy(src_h.at[pl.ds(off, C)], sidx)
                        pltpu.sync_copy(dst_h.at[pl.ds(off, C)], didx)
                        pltpu.async_copy(tbl_h.at[sidx], rows, gsem).wait()
                        pltpu.sync_copy(rows, acc.at[didx], add=True)
                        return carry

                    lax.fori_loop(0, NITER, body, 0)
                    plsc.subcore_barrier()
                    for j in range(NDR):
                        r0 = s * RPT + j * DR
                        pltpu.sync_copy(acc.at[pl.ds(r0, DR)], dbuf)
                        pltpu.sync_copy(dbuf, out_h.at[pl.ds(r0, DR)])

    return k(*tables, src, dst, z32)


def _dense1(a0, a1, c0, c1, x, W1l, b1l, W1r):
    def body(a0_r, a1_r, c0_r, c1_r, x_r, wl_r, bl_r, wr_r,
             o0, o1, o2, o3, rc):
        cnt = c0_r[:, 0:1] + c1_r[:, 0:1]
        recip = 1.0 / jnp.maximum(cnt, 1.0)
        m = lax.dot_general(a0_r[...], wl_r[:, :W],
                            (((1,), (1,)), ((), ())),
                            preferred_element_type=jnp.float32)
        m += lax.dot_general(a1_r[...], wl_r[:, W:],
                             (((1,), (1,)), ((), ())),
                             preferred_element_type=jnp.float32)
        sf = lax.dot_general(x_r[...], wr_r[...],
                             (((1,), (1,)), ((), ())),
                             preferred_element_type=jnp.float32)
        h = jnp.maximum(m * recip + bl_r[...] + sf, 0.0)
        o0[...] = h[:, 0:W]
        o1[...] = h[:, W:2 * W]
        o2[...] = h[:, 2 * W:3 * W]
        o3[...] = h[:, 3 * W:4 * W]
        rc[...] = jnp.broadcast_to(recip, (BN, 8))

    node = lambda w: pl.BlockSpec((BN, w), lambda i: (i, 0))
    full = lambda a, b: pl.BlockSpec((a, b), lambda i: (0, 0))
    return pl.pallas_call(
        body,
        grid=(GRID,),
        in_specs=[node(W), node(W), node(CW), node(CW), node(D),
                  full(H, D), full(1, H), full(H, D)],
        out_specs=[node(W), node(W), node(W), node(W), node(8)],
        out_shape=[jax.ShapeDtypeStruct((NP, W), jnp.float32) for _ in range(4)]
        + [jax.ShapeDtypeStruct((NP, 8), jnp.float32)],
    )(a0, a1, c0, c1, x, W1l, b1l, W1r)


def _dense2(g0, g1, g2, g3, h0, h1, h2, h3, rc, W2l, b2l, W2r, Wc, bc):
    def body(g0_r, g1_r, g2_r, g3_r, h0_r, h1_r, h2_r, h3_r, rc_r,
             wl_r, bl_r, wr_r, wc_r, bc_r, o):
        gs = (g0_r, g1_r, g2_r, g3_r)
        hs = (h0_r, h1_r, h2_r, h3_r)
        m = jnp.zeros((BN, H), jnp.float32)
        sf = jnp.zeros((BN, H), jnp.float32)
        for i in range(4):
            m += lax.dot_general(gs[i][...], wl_r[:, i * W:(i + 1) * W],
                                 (((1,), (1,)), ((), ())),
                                 preferred_element_type=jnp.float32)
            sf += lax.dot_general(hs[i][...], wr_r[:, i * W:(i + 1) * W],
                                  (((1,), (1,)), ((), ())),
                                  preferred_element_type=jnp.float32)
        h = jnp.maximum(m * rc_r[:, 0:1] + bl_r[...] + sf, 0.0)
        logit = jnp.sum(h * wc_r[...], axis=1, keepdims=True) + bc_r[0]
        o[...] = 1.0 / (1.0 + jnp.exp(-logit))

    node = lambda w: pl.BlockSpec((BN, w), lambda i: (i, 0))
    full = lambda a, b: pl.BlockSpec((a, b), lambda i: (0, 0))
    return pl.pallas_call(
        body,
        grid=(GRID,),
        in_specs=[node(W)] * 8 + [node(8),
                  full(H, H), full(1, H), full(H, H), full(1, H),
                  pl.BlockSpec(memory_space=pltpu.SMEM)],
        out_specs=node(1),
        out_shape=jax.ShapeDtypeStruct((NP, 1), jnp.float32),
    )(g0, g1, g2, g3, h0, h1, h2, h3, rc, W2l, b2l, W2r, Wc, bc)


def kernel(x, edge_index, W1l, b1l, W1r, W2l, b2l, W2r, Wc, bc):
    src = edge_index[0]
    dst = edge_index[1]
    t0 = x[:, :W]
    t1 = x[:, W:]
    z32 = jnp.zeros((DR, W), jnp.float32)
    z8 = jnp.zeros((DR, CW), jnp.float32)
    o8 = jnp.ones((CC_, CW), jnp.float32)

    cnt = _counts(dst, z8, o8)
    a0, a1 = _agg([t0, t1], src, dst, z32)
    h0, h1, h2, h3, rc = _dense1(a0, a1, cnt[0], cnt[1], x,
                                 W1l, b1l.reshape(1, H), W1r)
    g0, g1, g2, g3 = _agg([h0, h1, h2, h3], src, dst, z32)
    out = _dense2(g0, g1, g2, g3, h0, h1, h2, h3, rc,
                  W2l, b2l.reshape(1, H), W2r, Wc, bc.reshape(1))
    return out[:N]


# speedup vs baseline: 6.1737x; 2.7580x over previous
import functools

import jax
import jax.numpy as jnp
from jax import lax
from jax.experimental import pallas as pl
from jax.experimental.pallas import tpu as pltpu
from jax.experimental.pallas import tpu_sc as plsc

N = 50000
NP = 50176
E = 800000
D = 64
H = 128
W = 32

NC = 2
NS = 16
C = 80
EPT = E // NS
NITER = EPT // C
RPT = NP // NS
DR = 112
NDR = RPT // DR

CW = 8
CC_ = 40
CK = 5
CEPW = E // (NC * NS)
CNG = CEPW // (CC_ * CK)

K = 8
NG = (NITER - 1) // K

BN = 784
GRID = NP // BN

_mesh = lambda: plsc.VectorSubcoreMesh(core_axis_name="c", subcore_axis_name="s")


def _counts(dst, z8, o8):
    @functools.partial(
        pl.kernel,
        mesh=_mesh(),
        compiler_params=pltpu.CompilerParams(use_tc_tiling_on_sc=False),
        out_type=jax.ShapeDtypeStruct((NC, NP, CW), jnp.float32),
        scratch_types=[
            pltpu.VMEM((CK, CC_), jnp.int32),
            pltpu.VMEM((CC_, CW), jnp.float32),
            pltpu.VMEM((DR, CW), jnp.float32),
            pltpu.VMEM_SHARED((NP, CW), jnp.float32),
            pltpu.SemaphoreType.DMA,
        ],
    )
    def k(dstc_h, z8_h, o8_h, cnt_h, cidx, ones, cbuf, cacc, ssem):
        c = lax.axis_index("c")
        s = lax.axis_index("s")
        pltpu.sync_copy(o8_h, ones)
        pltpu.sync_copy(z8_h, cbuf)
        for j in range(NDR):
            pltpu.sync_copy(cbuf, cacc.at[pl.ds(s * RPT + j * DR, DR)])
        plsc.subcore_barrier()
        for cc in range(NC):
            @pl.when(c == cc)
            def _():
                def body(g, carry):
                    pltpu.sync_copy(dstc_h.at[s, cc, g], cidx)
                    sds = [pltpu.async_copy(ones, cacc.at[cidx.at[j]],
                                            ssem, add=True)
                           for j in range(CK)]
                    for d in sds:
                        d.wait()
                    return carry

                lax.fori_loop(0, CNG, body, 0)
                plsc.subcore_barrier()
                for j in range(NDR):
                    r0 = s * RPT + j * DR
                    pltpu.sync_copy(cacc.at[pl.ds(r0, DR)], cbuf)
                    pltpu.sync_copy(cbuf, cnt_h.at[cc, pl.ds(r0, DR)])

    return k(dst.reshape(NS, NC, CNG, CK, CC_), z8, o8)


def _agg(tables, src, dst, z32):
    ntab = len(tables)
    rounds = ntab // NC

    @functools.partial(
        pl.kernel,
        mesh=_mesh(),
        compiler_params=pltpu.CompilerParams(use_tc_tiling_on_sc=False),
        out_type=[jax.ShapeDtypeStruct((NP, W), jnp.float32)
                  for _ in range(ntab)],
        scratch_types=[
            pltpu.VMEM((K, C), jnp.int32),
            pltpu.VMEM((K, C), jnp.int32),
            pltpu.VMEM((K, C, W), jnp.float32),
            pltpu.VMEM((DR, W), jnp.float32),
            pltpu.VMEM_SHARED((NP, W), jnp.float32),
            pltpu.SemaphoreType.DMA((K,)),
            pltpu.SemaphoreType.DMA,
        ],
    )
    def k(*refs):
        tbls = refs[:ntab]
        src_h, dst_h, z32_h = refs[ntab:ntab + 3]
        outs = refs[ntab + 3:2 * ntab + 3]
        sidx2, didx2, rows2, dbuf, acc, gsem, ssem = refs[2 * ntab + 3:]
        c = lax.axis_index("c")
        s = lax.axis_index("s")
        for cc in range(NC):
            @pl.when(c == cc)
            def _():
                for rnd in range(rounds):
                    ci = cc + NC * rnd
                    tbl_h = tbls[ci]
                    out_h = outs[ci]
                    pltpu.sync_copy(z32_h, dbuf)
                    for j in range(NDR):
                        pltpu.sync_copy(dbuf, acc.at[pl.ds(s * RPT + j * DR, DR)])
                    plsc.subcore_barrier()

                    def group(g, carry):
                        pltpu.sync_copy(src_h.at[s, pl.ds(K * g, K)], sidx2)
                        pltpu.sync_copy(dst_h.at[s, pl.ds(K * g, K)], didx2)
                        gds = [pltpu.async_copy(tbl_h.at[sidx2.at[j]],
                                                rows2.at[j], gsem.at[j])
                               for j in range(K)]
                        sds = []
                        for j in range(K):
                            gds[j].wait()
                            sds.append(pltpu.async_copy(
                                rows2.at[j], acc.at[didx2.at[j]],
                                ssem, add=True))
                        for d in sds:
                            d.wait()
                        return carry

                    lax.fori_loop(0, NG, group, 0)
                    pltpu.sync_copy(src_h.at[s, pl.ds(K * NG, 1)],
                                    sidx2.at[pl.ds(0, 1)])
                    pltpu.sync_copy(dst_h.at[s, pl.ds(K * NG, 1)],
                                    didx2.at[pl.ds(0, 1)])
                    pltpu.async_copy(tbl_h.at[sidx2.at[0]], rows2.at[0],
                                     gsem.at[0]).wait()
                    pltpu.sync_copy(rows2.at[0], acc.at[didx2.at[0]], add=True)
                    plsc.subcore_barrier()
                    for j in range(NDR):
                        r0 = s * RPT + j * DR
                        pltpu.sync_copy(acc.at[pl.ds(r0, DR)], dbuf)
                        pltpu.sync_copy(dbuf, out_h.at[pl.ds(r0, DR)])

    return k(*tables, src.reshape(NS, NITER, C), dst.reshape(NS, NITER, C), z32)


def _dense1(a0, a1, c0, c1, x, W1l, b1l, W1r):
    def body(a0_r, a1_r, c0_r, c1_r, x_r, wl_r, bl_r, wr_r,
             o0, o1, o2, o3, rc):
        cnt = c0_r[:, 0:1] + c1_r[:, 0:1]
        recip = 1.0 / jnp.maximum(cnt, 1.0)
        m = lax.dot_general(a0_r[...], wl_r[:, :W],
                            (((1,), (1,)), ((), ())),
                            preferred_element_type=jnp.float32)
        m += lax.dot_general(a1_r[...], wl_r[:, W:],
                             (((1,), (1,)), ((), ())),
                             preferred_element_type=jnp.float32)
        sf = lax.dot_general(x_r[...], wr_r[...],
                             (((1,), (1,)), ((), ())),
                             preferred_element_type=jnp.float32)
        h = jnp.maximum(m * recip + bl_r[...] + sf, 0.0)
        o0[...] = h[:, 0:W]
        o1[...] = h[:, W:2 * W]
        o2[...] = h[:, 2 * W:3 * W]
        o3[...] = h[:, 3 * W:4 * W]
        rc[...] = jnp.broadcast_to(recip, (BN, 8))

    node = lambda w: pl.BlockSpec((BN, w), lambda i: (i, 0))
    full = lambda a, b: pl.BlockSpec((a, b), lambda i: (0, 0))
    return pl.pallas_call(
        body,
        grid=(GRID,),
        in_specs=[node(W), node(W), node(CW), node(CW), node(D),
                  full(H, D), full(1, H), full(H, D)],
        out_specs=[node(W), node(W), node(W), node(W), node(8)],
        out_shape=[jax.ShapeDtypeStruct((NP, W), jnp.float32) for _ in range(4)]
        + [jax.ShapeDtypeStruct((NP, 8), jnp.float32)],
    )(a0, a1, c0, c1, x, W1l, b1l, W1r)


def _dense2(g0, g1, g2, g3, h0, h1, h2, h3, rc, W2l, b2l, W2r, Wc, bc):
    def body(g0_r, g1_r, g2_r, g3_r, h0_r, h1_r, h2_r, h3_r, rc_r,
             wl_r, bl_r, wr_r, wc_r, bc_r, o):
        gs = (g0_r, g1_r, g2_r, g3_r)
        hs = (h0_r, h1_r, h2_r, h3_r)
        m = jnp.zeros((BN, H), jnp.float32)
        sf = jnp.zeros((BN, H), jnp.float32)
        for i in range(4):
            m += lax.dot_general(gs[i][...], wl_r[:, i * W:(i + 1) * W],
                                 (((1,), (1,)), ((), ())),
                                 preferred_element_type=jnp.float32)
            sf += lax.dot_general(hs[i][...], wr_r[:, i * W:(i + 1) * W],
                                  (((1,), (1,)), ((), ())),
                                  preferred_element_type=jnp.float32)
        h = jnp.maximum(m * rc_r[:, 0:1] + bl_r[...] + sf, 0.0)
        logit = jnp.sum(h * wc_r[...], axis=1, keepdims=True) + bc_r[0]
        o[...] = 1.0 / (1.0 + jnp.exp(-logit))

    node = lambda w: pl.BlockSpec((BN, w), lambda i: (i, 0))
    full = lambda a, b: pl.BlockSpec((a, b), lambda i: (0, 0))
    return pl.pallas_call(
        body,
        grid=(GRID,),
        in_specs=[node(W)] * 8 + [node(8),
                  full(H, H), full(1, H), full(H, H), full(1, H),
                  pl.BlockSpec(memory_space=pltpu.SMEM)],
        out_specs=node(1),
        out_shape=jax.ShapeDtypeStruct((NP, 1), jnp.float32),
    )(g0, g1, g2, g3, h0, h1, h2, h3, rc, W2l, b2l, W2r, Wc, bc)


def kernel(x, edge_index, W1l, b1l, W1r, W2l, b2l, W2r, Wc, bc):
    src = edge_index[0]
    dst = edge_index[1]
    t0 = x[:, :W]
    t1 = x[:, W:]
    z32 = jnp.zeros((DR, W), jnp.float32)
    z8 = jnp.zeros((DR, CW), jnp.float32)
    o8 = jnp.ones((CC_, CW), jnp.float32)

    cnt = _counts(dst, z8, o8)
    a0, a1 = _agg([t0, t1], src, dst, z32)
    h0, h1, h2, h3, rc = _dense1(a0, a1, cnt[0], cnt[1], x,
                                 W1l, b1l.reshape(1, H), W1r)
    g0, g1, g2, g3 = _agg([h0, h1, h2, h3], src, dst, z32)
    out = _dense2(g0, g1, g2, g3, h0, h1, h2, h3, rc,
                  W2l, b2l.reshape(1, H), W2r, Wc, bc.reshape(1))
    return out[:N]


# speedup vs baseline: 7.0198x; 1.1371x over previous
import functools

import jax
import jax.numpy as jnp
from jax import lax
from jax.experimental import pallas as pl
from jax.experimental.pallas import tpu as pltpu
from jax.experimental.pallas import tpu_sc as plsc

N = 50000
NP = 50176
E = 800000
D = 64
H = 128
W = 32

NC = 2
NS = 16
C = 80
EPT = E // NS
NITER = EPT // C
RPT = NP // NS
DR = 112
NDR = RPT // DR

CW = 8
CC_ = 40
CK = 5
CEPW = E // (NC * NS)
CNG = CEPW // (CC_ * CK)

K = 8
NG = (NITER - 1) // K

BN = 784
GRID = NP // BN

_mesh = lambda: plsc.VectorSubcoreMesh(core_axis_name="c", subcore_axis_name="s")


def _counts(dst, z8, o8):
    @functools.partial(
        pl.kernel,
        mesh=_mesh(),
        compiler_params=pltpu.CompilerParams(use_tc_tiling_on_sc=False),
        out_type=jax.ShapeDtypeStruct((NC, NP, CW), jnp.float32),
        scratch_types=[
            pltpu.VMEM((CK, CC_), jnp.int32),
            pltpu.VMEM((CC_, CW), jnp.float32),
            pltpu.VMEM((DR, CW), jnp.float32),
            pltpu.VMEM_SHARED((NP, CW), jnp.float32),
            pltpu.SemaphoreType.DMA,
        ],
    )
    def k(dstc_h, z8_h, o8_h, cnt_h, cidx, ones, cbuf, cacc, ssem):
        c = lax.axis_index("c")
        s = lax.axis_index("s")
        pltpu.sync_copy(o8_h, ones)
        pltpu.sync_copy(z8_h, cbuf)
        for j in range(NDR):
            pltpu.sync_copy(cbuf, cacc.at[pl.ds(s * RPT + j * DR, DR)])
        plsc.subcore_barrier()
        for cc in range(NC):
            @pl.when(c == cc)
            def _():
                def body(g, carry):
                    pltpu.sync_copy(dstc_h.at[s, cc, g], cidx)
                    sds = [pltpu.async_copy(ones, cacc.at[cidx.at[j]],
                                            ssem, add=True)
                           for j in range(CK)]
                    for d in sds:
                        d.wait()
                    return carry

                lax.fori_loop(0, CNG, body, 0)
                plsc.subcore_barrier()
                for j in range(NDR):
                    r0 = s * RPT + j * DR
                    pltpu.sync_copy(cacc.at[pl.ds(r0, DR)], cbuf)
                    pltpu.sync_copy(cbuf, cnt_h.at[cc, pl.ds(r0, DR)])

    return k(dst.reshape(NS, NC, CNG, CK, CC_), z8, o8)


def _agg(tables, nchunks, src, dst, z32):
    ntab = len(tables)
    rounds = nchunks // NC

    @functools.partial(
        pl.kernel,
        mesh=_mesh(),
        compiler_params=pltpu.CompilerParams(use_tc_tiling_on_sc=False),
        out_type=jax.ShapeDtypeStruct((NP, H), jnp.float32),
        scratch_types=[
            pltpu.VMEM((K, C), jnp.int32),
            pltpu.VMEM((K, C), jnp.int32),
            pltpu.VMEM((K, C, W), jnp.float32),
            pltpu.VMEM((DR, W), jnp.float32),
            pltpu.VMEM_SHARED((NP, W), jnp.float32),
            pltpu.SemaphoreType.DMA((K,)),
            pltpu.SemaphoreType.DMA,
        ],
    )
    def k(*refs):
        tbls = refs[:ntab]
        src_h, dst_h, z32_h = refs[ntab:ntab + 3]
        out_h = refs[ntab + 3]
        sidx2, didx2, rows2, dbuf, acc, gsem, ssem = refs[ntab + 4:]
        c = lax.axis_index("c")
        s = lax.axis_index("s")
        for cc in range(NC):
            @pl.when(c == cc)
            def _():
                for rnd in range(rounds):
                    ci = cc + NC * rnd
                    if ntab == nchunks:
                        gtbl = lambda idx, _ci=ci: tbls[_ci].at[idx]
                    else:
                        gtbl = lambda idx, _ci=ci: tbls[0].at[idx, pl.ds(_ci * W, W)]
                    pltpu.sync_copy(z32_h, dbuf)
                    for j in range(NDR):
                        pltpu.sync_copy(dbuf, acc.at[pl.ds(s * RPT + j * DR, DR)])
                    plsc.subcore_barrier()

                    def group(g, carry):
                        pltpu.sync_copy(src_h.at[s, pl.ds(K * g, K)], sidx2)
                        pltpu.sync_copy(dst_h.at[s, pl.ds(K * g, K)], didx2)
                        gds = [pltpu.async_copy(gtbl(sidx2.at[j]),
                                                rows2.at[j], gsem.at[j])
                               for j in range(K)]
                        sds = []
                        for j in range(K):
                            gds[j].wait()
                            sds.append(pltpu.async_copy(
                                rows2.at[j], acc.at[didx2.at[j]],
                                ssem, add=True))
                        for d in sds:
                            d.wait()
                        return carry

                    lax.fori_loop(0, NG, group, 0)
                    pltpu.sync_copy(src_h.at[s, pl.ds(K * NG, 1)],
                                    sidx2.at[pl.ds(0, 1)])
                    pltpu.sync_copy(dst_h.at[s, pl.ds(K * NG, 1)],
                                    didx2.at[pl.ds(0, 1)])
                    pltpu.async_copy(gtbl(sidx2.at[0]), rows2.at[0],
                                     gsem.at[0]).wait()
                    pltpu.sync_copy(rows2.at[0], acc.at[didx2.at[0]], add=True)
                    plsc.subcore_barrier()
                    for j in range(NDR):
                        r0 = s * RPT + j * DR
                        pltpu.sync_copy(acc.at[pl.ds(r0, DR)], dbuf)
                        pltpu.sync_copy(
                            dbuf, out_h.at[pl.ds(r0, DR), pl.ds(ci * W, W)])

    return k(*tables, src.reshape(NS, NITER, C), dst.reshape(NS, NITER, C), z32)


def _dense1(af, c0, c1, x, W1l, b1l, W1r):
    def body(af_r, c0_r, c1_r, x_r, wl_r, bl_r, wr_r, o):
        cnt = c0_r[:, 0:1] + c1_r[:, 0:1]
        recip = 1.0 / jnp.maximum(cnt, 1.0)
        m = lax.dot_general(af_r[:, :D], wl_r[...],
                            (((1,), (1,)), ((), ())),
                            preferred_element_type=jnp.float32)
        sf = lax.dot_general(x_r[...], wr_r[...],
                             (((1,), (1,)), ((), ())),
                             preferred_element_type=jnp.float32)
        o[...] = jnp.maximum(m * recip + bl_r[...] + sf, 0.0)

    node = lambda w: pl.BlockSpec((BN, w), lambda i: (i, 0))
    full = lambda a, b: pl.BlockSpec((a, b), lambda i: (0, 0))
    return pl.pallas_call(
        body,
        grid=(GRID,),
        in_specs=[node(H), node(CW), node(CW), node(D),
                  full(H, D), full(1, H), full(H, D)],
        out_specs=node(H),
        out_shape=jax.ShapeDtypeStruct((NP, H), jnp.float32),
    )(af, c0, c1, x, W1l, b1l, W1r)


def _dense2(gf, hf, c0, c1, W2l, b2l, W2r, Wc, bc):
    def body(gf_r, hf_r, c0_r, c1_r, wl_r, bl_r, wr_r, wc_r, bc_r, o):
        cnt = c0_r[:, 0:1] + c1_r[:, 0:1]
        recip = 1.0 / jnp.maximum(cnt, 1.0)
        m = lax.dot_general(gf_r[...], wl_r[...],
                            (((1,), (1,)), ((), ())),
                            preferred_element_type=jnp.float32)
        sf = lax.dot_general(hf_r[...], wr_r[...],
                             (((1,), (1,)), ((), ())),
                             preferred_element_type=jnp.float32)
        h = jnp.maximum(m * recip + bl_r[...] + sf, 0.0)
        logit = jnp.sum(h * wc_r[...], axis=1, keepdims=True) + bc_r[0]
        o[...] = 1.0 / (1.0 + jnp.exp(-logit))

    node = lambda w: pl.BlockSpec((BN, w), lambda i: (i, 0))
    full = lambda a, b: pl.BlockSpec((a, b), lambda i: (0, 0))
    return pl.pallas_call(
        body,
        grid=(GRID,),
        in_specs=[node(H), node(H), node(CW), node(CW),
                  full(H, H), full(1, H), full(H, H), full(1, H),
                  pl.BlockSpec(memory_space=pltpu.SMEM)],
        out_specs=node(1),
        out_shape=jax.ShapeDtypeStruct((NP, 1), jnp.float32),
    )(gf, hf, c0, c1, W2l, b2l, W2r, Wc, bc)


def kernel(x, edge_index, W1l, b1l, W1r, W2l, b2l, W2r, Wc, bc):
    src = edge_index[0]
    dst = edge_index[1]
    t0 = x[:, :W]
    t1 = x[:, W:]
    z32 = jnp.zeros((DR, W), jnp.float32)
    z8 = jnp.zeros((DR, CW), jnp.float32)
    o8 = jnp.ones((CC_, CW), jnp.float32)

    cnt = _counts(dst, z8, o8)
    af = _agg([t0, t1], 2, src, dst, z32)
    hf = _dense1(af, cnt[0], cnt[1], x, W1l, b1l.reshape(1, H), W1r)
    gf = _agg([hf[:, 0:W], hf[:, W:2 * W], hf[:, 2 * W:3 * W],
               hf[:, 3 * W:4 * W]], 4, src, dst, z32)
    out = _dense2(gf, hf, cnt[0], cnt[1],
                  W2l, b2l.reshape(1, H), W2r, Wc, bc.reshape(1))
    return out[:N]


# speedup vs baseline: 7.9517x; 1.1328x over previous
import functools

import jax
import jax.numpy as jnp
from jax import lax
from jax.experimental import pallas as pl
from jax.experimental.pallas import tpu as pltpu
from jax.experimental.pallas import tpu_sc as plsc

N = 50000
NP = 50176
E = 800000
D = 64
H = 128
W = 32

NC = 2
NS = 16
C = 80
EPT = E // NS
NITER = EPT // C
RPT = NP // NS
DR = 112
NDR = RPT // DR

CW = 8
CC_ = 40
CK = 5
CEPW = E // (NC * NS)
CNG = CEPW // (CC_ * CK)

K = 8
NG = (NITER - 1) // K

BN = 784
GRID = NP // BN

_mesh = lambda: plsc.VectorSubcoreMesh(core_axis_name="c", subcore_axis_name="s")


def _counts(dst, z8, o8):
    @functools.partial(
        pl.kernel,
        mesh=_mesh(),
        compiler_params=pltpu.CompilerParams(use_tc_tiling_on_sc=False),
        out_type=jax.ShapeDtypeStruct((NC, NP, CW), jnp.float32),
        scratch_types=[
            pltpu.VMEM((CK, CC_), jnp.int32),
            pltpu.VMEM((CC_, CW), jnp.float32),
            pltpu.VMEM((DR, CW), jnp.float32),
            pltpu.VMEM_SHARED((NP, CW), jnp.float32),
            pltpu.SemaphoreType.DMA,
        ],
    )
    def k(dstc_h, z8_h, o8_h, cnt_h, cidx, ones, cbuf, cacc, ssem):
        c = lax.axis_index("c")
        s = lax.axis_index("s")
        pltpu.sync_copy(o8_h, ones)
        pltpu.sync_copy(z8_h, cbuf)
        for j in range(NDR):
            pltpu.sync_copy(cbuf, cacc.at[pl.ds(s * RPT + j * DR, DR)])
        plsc.subcore_barrier()
        for cc in range(NC):
            @pl.when(c == cc)
            def _():
                def body(g, carry):
                    pltpu.sync_copy(dstc_h.at[s, cc, g], cidx)
                    sds = [pltpu.async_copy(ones, cacc.at[cidx.at[j]],
                                            ssem, add=True)
                           for j in range(CK)]
                    for d in sds:
                        d.wait()
                    return carry

                lax.fori_loop(0, CNG, body, 0)
                plsc.subcore_barrier()
                for j in range(NDR):
                    r0 = s * RPT + j * DR
                    pltpu.sync_copy(cacc.at[pl.ds(r0, DR)], cbuf)
                    pltpu.sync_copy(cbuf, cnt_h.at[cc, pl.ds(r0, DR)])

    return k(dst.reshape(NS, NC, CNG, CK, CC_), z8, o8)


def _agg(table, F, nchunks, src, dst, z32):
    rounds = nchunks // NC

    @functools.partial(
        pl.kernel,
        mesh=_mesh(),
        compiler_params=pltpu.CompilerParams(use_tc_tiling_on_sc=False),
        out_type=jax.ShapeDtypeStruct((NP, H), jnp.float32),
        scratch_types=[
            pltpu.VMEM((K, C), jnp.int32),
            pltpu.VMEM((K, C), jnp.int32),
            pltpu.VMEM((K, C), jnp.int32),
            pltpu.VMEM((K, C, W), jnp.float32),
            pltpu.VMEM((DR, W), jnp.float32),
            pltpu.VMEM_SHARED((NP, W), jnp.float32),
            pltpu.SemaphoreType.DMA((K,)),
            pltpu.SemaphoreType.DMA,
        ],
    )
    def k(tbl_h, src_h, dst_h, z32_h, out_h,
          sidx2, didx2, sadj, rows2, dbuf, acc, gsem, ssem):
        c = lax.axis_index("c")
        s = lax.axis_index("s")

        def adjust(nrows, ci):
            for j in range(nrows):
                for u in range(C // 16):
                    v = sidx2[j, pl.ds(u * 16, 16)]
                    sadj[j, pl.ds(u * 16, 16)] = v * F + ci

        for cc in range(NC):
            @pl.when(c == cc)
            def _():
                for rnd in range(rounds):
                    ci = cc + NC * rnd
                    pltpu.sync_copy(z32_h, dbuf)
                    for j in range(NDR):
                        pltpu.sync_copy(dbuf, acc.at[pl.ds(s * RPT + j * DR, DR)])
                    plsc.subcore_barrier()

                    def group(g, carry):
                        pltpu.sync_copy(src_h.at[s, pl.ds(K * g, K)], sidx2)
                        pltpu.sync_copy(dst_h.at[s, pl.ds(K * g, K)], didx2)
                        adjust(K, ci)
                        gds = [pltpu.async_copy(tbl_h.at[sadj.at[j]],
                                                rows2.at[j], gsem.at[j])
                               for j in range(K)]
                        sds = []
                        for j in range(K):
                            gds[j].wait()
                            sds.append(pltpu.async_copy(
                                rows2.at[j], acc.at[didx2.at[j]],
                                ssem, add=True))
                        for d in sds:
                            d.wait()
                        return carry

                    lax.fori_loop(0, NG, group, 0)
                    pltpu.sync_copy(src_h.at[s, pl.ds(K * NG, 1)],
                                    sidx2.at[pl.ds(0, 1)])
                    pltpu.sync_copy(dst_h.at[s, pl.ds(K * NG, 1)],
                                    didx2.at[pl.ds(0, 1)])
                    adjust(1, ci)
                    pltpu.async_copy(tbl_h.at[sadj.at[0]], rows2.at[0],
                                     gsem.at[0]).wait()
                    pltpu.sync_copy(rows2.at[0], acc.at[didx2.at[0]], add=True)
                    plsc.subcore_barrier()
                    for j in range(NDR):
                        r0 = s * RPT + j * DR
                        pltpu.sync_copy(acc.at[pl.ds(r0, DR)], dbuf)
                        pltpu.sync_copy(
                            dbuf, out_h.at[pl.ds(r0, DR), pl.ds(ci * W, W)])

    return k(table, src.reshape(NS, NITER, C), dst.reshape(NS, NITER, C), z32)


def _dense1(af, c0, c1, x, W1l, b1l, W1r):
    def body(af_r, c0_r, c1_r, x_r, wl_r, bl_r, wr_r, o):
        cnt = c0_r[:, 0:1] + c1_r[:, 0:1]
        recip = 1.0 / jnp.maximum(cnt, 1.0)
        m = lax.dot_general(af_r[:, :D], wl_r[...],
                            (((1,), (1,)), ((), ())),
                            preferred_element_type=jnp.float32)
        sf = lax.dot_general(x_r[...], wr_r[...],
                             (((1,), (1,)), ((), ())),
                             preferred_element_type=jnp.float32)
        o[...] = jnp.maximum(m * recip + bl_r[...] + sf, 0.0)

    node = lambda w: pl.BlockSpec((BN, w), lambda i: (i, 0))
    full = lambda a, b: pl.BlockSpec((a, b), lambda i: (0, 0))
    return pl.pallas_call(
        body,
        grid=(GRID,),
        in_specs=[node(H), node(CW), node(CW), node(D),
                  full(H, D), full(1, H), full(H, D)],
        out_specs=node(H),
        out_shape=jax.ShapeDtypeStruct((NP, H), jnp.float32),
    )(af, c0, c1, x, W1l, b1l, W1r)


def _dense2(gf, hf, c0, c1, W2l, b2l, W2r, Wc, bc):
    def body(gf_r, hf_r, c0_r, c1_r, wl_r, bl_r, wr_r, wc_r, bc_r, o):
        cnt = c0_r[:, 0:1] + c1_r[:, 0:1]
        recip = 1.0 / jnp.maximum(cnt, 1.0)
        m = lax.dot_general(gf_r[...], wl_r[...],
                            (((1,), (1,)), ((), ())),
                            preferred_element_type=jnp.float32)
        sf = lax.dot_general(hf_r[...], wr_r[...],
                             (((1,), (1,)), ((), ())),
                             preferred_element_type=jnp.float32)
        h = jnp.maximum(m * recip + bl_r[...] + sf, 0.0)
        logit = jnp.sum(h * wc_r[...], axis=1, keepdims=True) + bc_r[0]
        o[...] = 1.0 / (1.0 + jnp.exp(-logit))

    node = lambda w: pl.BlockSpec((BN, w), lambda i: (i, 0))
    full = lambda a, b: pl.BlockSpec((a, b), lambda i: (0, 0))
    return pl.pallas_call(
        body,
        grid=(GRID,),
        in_specs=[node(H), node(H), node(CW), node(CW),
                  full(H, H), full(1, H), full(H, H), full(1, H),
                  pl.BlockSpec(memory_space=pltpu.SMEM)],
        out_specs=node(1),
        out_shape=jax.ShapeDtypeStruct((NP, 1), jnp.float32),
    )(gf, hf, c0, c1, W2l, b2l, W2r, Wc, bc)


def kernel(x, edge_index, W1l, b1l, W1r, W2l, b2l, W2r, Wc, bc):
    src = edge_index[0]
    dst = edge_index[1]
    z32 = jnp.zeros((DR, W), jnp.float32)
    z8 = jnp.zeros((DR, CW), jnp.float32)
    o8 = jnp.ones((CC_, CW), jnp.float32)

    cnt = _counts(dst, z8, o8)
    af = _agg(x.reshape(N * 2, W), 2, 2, src, dst, z32)
    hf = _dense1(af, cnt[0], cnt[1], x, W1l, b1l.reshape(1, H), W1r)
    gf = _agg(hf.reshape(NP * 4, W), 4, 4, src, dst, z32)
    out = _dense2(gf, hf, cnt[0], cnt[1],
                  W2l, b2l.reshape(1, H), W2r, Wc, bc.reshape(1))
    return out[:N]


# speedup vs baseline: 9.0747x; 1.1412x over previous
import functools

import jax
import jax.numpy as jnp
from jax import lax
from jax.experimental import pallas as pl
from jax.experimental.pallas import tpu as pltpu
from jax.experimental.pallas import tpu_sc as plsc

N = 50000
NP = 50176
E = 800000
D = 64
H = 128
W = 32

NC = 2
NS = 16
C = 80
EPT = E // NS
NITER = EPT // C
RPT = NP // NS
DR = 112
NDR = RPT // DR

CW = 8
CC_ = 40
CK = 5
CEPW = E // (NC * NS)
CNG = CEPW // (CC_ * CK)

K = 8
NG = (NITER - 1) // K

BN = 784
GRID = NP // BN

_mesh = lambda: plsc.VectorSubcoreMesh(core_axis_name="c", subcore_axis_name="s")


def _counts(dst, z8, o8):
    @functools.partial(
        pl.kernel,
        mesh=_mesh(),
        compiler_params=pltpu.CompilerParams(use_tc_tiling_on_sc=False),
        out_type=jax.ShapeDtypeStruct((NC, NP, CW), jnp.float32),
        scratch_types=[
            pltpu.VMEM((CK, CC_), jnp.int32),
            pltpu.VMEM((CC_, CW), jnp.float32),
            pltpu.VMEM((DR, CW), jnp.float32),
            pltpu.VMEM_SHARED((NP, CW), jnp.float32),
            pltpu.SemaphoreType.DMA,
        ],
    )
    def k(dstc_h, z8_h, o8_h, cnt_h, cidx, ones, cbuf, cacc, ssem):
        c = lax.axis_index("c")
        s = lax.axis_index("s")
        pltpu.sync_copy(o8_h, ones)
        pltpu.sync_copy(z8_h, cbuf)
        for j in range(NDR):
            pltpu.sync_copy(cbuf, cacc.at[pl.ds(s * RPT + j * DR, DR)])
        plsc.subcore_barrier()
        for cc in range(NC):
            @pl.when(c == cc)
            def _():
                def body(g, carry):
                    pltpu.sync_copy(dstc_h.at[s, cc, g], cidx)
                    sds = [pltpu.async_copy(ones, cacc.at[cidx.at[j]],
                                            ssem, add=True)
                           for j in range(CK)]
                    for d in sds:
                        d.wait()
                    return carry

                lax.fori_loop(0, CNG, body, 0)
                plsc.subcore_barrier()
                for j in range(NDR):
                    r0 = s * RPT + j * DR
                    pltpu.sync_copy(cacc.at[pl.ds(r0, DR)], cbuf)
                    pltpu.sync_copy(cbuf, cnt_h.at[cc, pl.ds(r0, DR)])

    return k(dst.reshape(NS, NC, CNG, CK, CC_), z8, o8)


def _agg(table, F, nchunks, src, dst, z32):
    rounds = nchunks // NC

    @functools.partial(
        pl.kernel,
        mesh=_mesh(),
        compiler_params=pltpu.CompilerParams(use_tc_tiling_on_sc=False),
        out_type=jax.ShapeDtypeStruct((NP, H), jnp.float32),
        scratch_types=[
            pltpu.VMEM((2, K, C), jnp.int32),
            pltpu.VMEM((2, K, C), jnp.int32),
            pltpu.VMEM((K, C), jnp.int32),
            pltpu.VMEM((K, C, W), jnp.float32),
            pltpu.VMEM((DR, W), jnp.float32),
            pltpu.VMEM_SHARED((NP, W), jnp.float32),
            pltpu.SemaphoreType.DMA((K,)),
            pltpu.SemaphoreType.DMA((K,)),
        ],
    )
    def k(tbl_h, src_h, dst_h, z32_h, out_h,
          sidx2, didx2, sadj, rows2, dbuf, acc, gsem, ssem):
        c = lax.axis_index("c")
        s = lax.axis_index("s")

        def adjust(p, ci):
            for j in range(K):
                for u in range(C // 16):
                    v = sidx2[p, j, pl.ds(u * 16, 16)]
                    sadj[j, pl.ds(u * 16, 16)] = v * F + ci

        def wait_slot(j, prev_p):
            pltpu.make_async_copy(rows2.at[j], acc.at[didx2.at[prev_p, j]],
                                  ssem.at[j]).wait()

        for cc in range(NC):
            @pl.when(c == cc)
            def _():
                for rnd in range(rounds):
                    ci = cc + NC * rnd
                    pltpu.sync_copy(z32_h, dbuf)
                    for j in range(NDR):
                        pltpu.sync_copy(dbuf, acc.at[pl.ds(s * RPT + j * DR, DR)])
                    plsc.subcore_barrier()

                    def gbody(g, p, first):
                        pltpu.sync_copy(src_h.at[s, pl.ds(K * g, K)],
                                        sidx2.at[p])
                        pltpu.sync_copy(dst_h.at[s, pl.ds(K * g, K)],
                                        didx2.at[p])
                        adjust(p, ci)
                        gds = []
                        for j in range(K):
                            if not first:
                                wait_slot(j, 1 - p)
                            gds.append(pltpu.async_copy(
                                tbl_h.at[sadj.at[j]], rows2.at[j], gsem.at[j]))
                        for j in range(K):
                            gds[j].wait()
                            pltpu.async_copy(rows2.at[j],
                                             acc.at[didx2.at[p, j]],
                                             ssem.at[j], add=True)

                    gbody(0, 0, True)

                    def pair(t, carry):
                        gbody(2 * t + 1, 1, False)
                        gbody(2 * t + 2, 0, False)
                        return carry

                    lax.fori_loop(0, (NG - 2) // 2, pair, 0)
                    gbody(NG - 1, 1, False)
                    for j in range(K):
                        wait_slot(j, 1)
                    pltpu.sync_copy(src_h.at[s, pl.ds(K * NG, 1)],
                                    sidx2.at[0, pl.ds(0, 1)])
                    pltpu.sync_copy(dst_h.at[s, pl.ds(K * NG, 1)],
                                    didx2.at[0, pl.ds(0, 1)])
                    for u in range(C // 16):
                        v = sidx2[0, 0, pl.ds(u * 16, 16)]
                        sadj[0, pl.ds(u * 16, 16)] = v * F + ci
                    pltpu.async_copy(tbl_h.at[sadj.at[0]], rows2.at[0],
                                     gsem.at[0]).wait()
                    pltpu.sync_copy(rows2.at[0], acc.at[didx2.at[0, 0]],
                                    add=True)
                    plsc.subcore_barrier()
                    for j in range(NDR):
                        r0 = s * RPT + j * DR
                        pltpu.sync_copy(acc.at[pl.ds(r0, DR)], dbuf)
                        pltpu.sync_copy(
                            dbuf, out_h.at[pl.ds(r0, DR), pl.ds(ci * W, W)])

    return k(table, src.reshape(NS, NITER, C), dst.reshape(NS, NITER, C), z32)


def _dense1(af, c0, c1, x, W1l, b1l, W1r):
    def body(af_r, c0_r, c1_r, x_r, wl_r, bl_r, wr_r, o):
        cnt = c0_r[:, 0:1] + c1_r[:, 0:1]
        recip = 1.0 / jnp.maximum(cnt, 1.0)
        m = lax.dot_general(af_r[:, :D], wl_r[...],
                            (((1,), (1,)), ((), ())),
                            preferred_element_type=jnp.float32)
        sf = lax.dot_general(x_r[...], wr_r[...],
                             (((1,), (1,)), ((), ())),
                             preferred_element_type=jnp.float32)
        o[...] = jnp.maximum(m * recip + bl_r[...] + sf, 0.0)

    node = lambda w: pl.BlockSpec((BN, w), lambda i: (i, 0))
    full = lambda a, b: pl.BlockSpec((a, b), lambda i: (0, 0))
    return pl.pallas_call(
        body,
        grid=(GRID,),
        in_specs=[node(H), node(CW), node(CW), node(D),
                  full(H, D), full(1, H), full(H, D)],
        out_specs=node(H),
        out_shape=jax.ShapeDtypeStruct((NP, H), jnp.float32),
    )(af, c0, c1, x, W1l, b1l, W1r)


def _dense2(gf, hf, c0, c1, W2l, b2l, W2r, Wc, bc):
    def body(gf_r, hf_r, c0_r, c1_r, wl_r, bl_r, wr_r, wc_r, bc_r, o):
        cnt = c0_r[:, 0:1] + c1_r[:, 0:1]
        recip = 1.0 / jnp.maximum(cnt, 1.0)
        m = lax.dot_general(gf_r[...], wl_r[...],
                            (((1,), (1,)), ((), ())),
                            preferred_element_type=jnp.float32)
        sf = lax.dot_general(hf_r[...], wr_r[...],
                             (((1,), (1,)), ((), ())),
                             preferred_element_type=jnp.float32)
        h = jnp.maximum(m * recip + bl_r[...] + sf, 0.0)
        logit = jnp.sum(h * wc_r[...], axis=1, keepdims=True) + bc_r[0]
        o[...] = 1.0 / (1.0 + jnp.exp(-logit))

    node = lambda w: pl.BlockSpec((BN, w), lambda i: (i, 0))
    full = lambda a, b: pl.BlockSpec((a, b), lambda i: (0, 0))
    return pl.pallas_call(
        body,
        grid=(GRID,),
        in_specs=[node(H), node(H), node(CW), node(CW),
                  full(H, H), full(1, H), full(H, H), full(1, H),
                  pl.BlockSpec(memory_space=pltpu.SMEM)],
        out_specs=node(1),
        out_shape=jax.ShapeDtypeStruct((NP, 1), jnp.float32),
    )(gf, hf, c0, c1, W2l, b2l, W2r, Wc, bc)


def kernel(x, edge_index, W1l, b1l, W1r, W2l, b2l, W2r, Wc, bc):
    src = edge_index[0]
    dst = edge_index[1]
    z32 = jnp.zeros((DR, W), jnp.float32)
    z8 = jnp.zeros((DR, CW), jnp.float32)
    o8 = jnp.ones((CC_, CW), jnp.float32)

    cnt = _counts(dst, z8, o8)
    af = _agg(x.reshape(N * 2, W), 2, 2, src, dst, z32)
    hf = _dense1(af, cnt[0], cnt[1], x, W1l, b1l.reshape(1, H), W1r)
    gf = _agg(hf.reshape(NP * 4, W), 4, 4, src, dst, z32)
    out = _dense2(gf, hf, cnt[0], cnt[1],
                  W2l, b2l.reshape(1, H), W2r, Wc, bc.reshape(1))
    return out[:N]
